# baseline (speedup 1.0000x reference)
import dataclasses

import jax
import jax.numpy as jnp
from jax import lax
from jax.experimental import pallas as pl
from jax.experimental.pallas import tpu as pltpu
from jax.experimental.pallas import tpu_sc as plsc

N_ROWS = 100000
D = 512
S = 256
NC = 2
NS = 16
NW = NC * NS
LANES = 16

RT = 59000
CH = 96
FH = 2
DHF = D // FH
FVEC = DHF // LANES
SEG_PER_W = S // NW
OFFS_PAD = 272

RB = 1000
NB = RT // RB
W = 32


def _sc_body(x_hbm, offs_hbm, out_hbm, xb0, xb1, offs_v, acc_v, sem0, sem1):
    c = lax.axis_index("c")
    s = lax.axis_index("s")
    w = c * NS + s

    zero = jnp.zeros((LANES,), jnp.float32)
    lane = lax.iota(jnp.int32, 16)

    pltpu.sync_copy(offs_hbm, offs_v)

    w0 = offs_v[pl.ds(w * SEG_PER_W, LANES)]
    w1 = offs_v[pl.ds(w * SEG_PER_W + 1, LANES)]

    def extract(vec, t):
        return jnp.maximum(jnp.sum(jnp.where(lane == t, vec, 0)), RT)

    seg_lo = [extract(w0, t) for t in range(SEG_PER_W)]
    seg_hi = [extract(w1, t) for t in range(SEG_PER_W)]
    row_start = seg_lo[0]
    row_end = seg_hi[SEG_PER_W - 1]

    @pl.loop(0, SEG_PER_W)
    def _(i):
        @pl.loop(0, FH * FVEC)
        def _(f):
            acc_v[i, pl.ds(f * LANES, LANES)] = zero

    a8 = (row_start // 8) * 8
    nk = (row_end - a8 + CH - 1) // CH

    def chunk_base(k):
        return jnp.minimum(a8 + k * CH, N_ROWS - CH)

    def start(k, xb, sem):
        pltpu.async_copy(x_hbm.at[pl.ds(chunk_base(k), CH), :], xb, sem)

    def wait(xb, sem):
        pltpu.make_async_copy(x_hbm.at[pl.ds(0, CH), :], xb, sem).wait()

    def process(k, xb):
        base = chunk_base(k)
        win_lo = jnp.maximum(row_start, a8 + k * CH)
        win_hi = jnp.minimum(row_end, a8 + (k + 1) * CH)
        for t in range(SEG_PER_W):
            a = jnp.maximum(seg_lo[t], win_lo) - base
            b = jnp.minimum(seg_hi[t], win_hi) - base
            for h in range(FH):
                def row_body(r, regs, h=h):
                    return tuple(
                        regs[f] + xb[r, pl.ds(h * DHF + f * LANES, LANES)]
                        for f in range(FVEC))

                regs0 = tuple(zero for _ in range(FVEC))
                regs = lax.fori_loop(a, b, row_body, regs0)
                for f in range(FVEC):
                    plsc.addupdate(
                        acc_v.at[t, pl.ds(h * DHF + f * LANES, LANES)],
                        regs[f])

    @pl.when(nk > 0)
    def _():
        start(0, xb0, sem0)

    def pair_body(m, carry):
        k0 = 2 * m

        @pl.when(k0 + 1 < nk)
        def _():
            start(k0 + 1, xb1, sem1)

        wait(xb0, sem0)
        process(k0, xb0)

        @pl.when(k0 + 2 < nk)
        def _():
            start(k0 + 2, xb0, sem0)

        @pl.when(k0 + 1 < nk)
        def _():
            wait(xb1, sem1)
            process(k0 + 1, xb1)

        return carry

    lax.fori_loop(0, (nk + 1) // 2, pair_body, 0)

    pltpu.sync_copy(acc_v, out_hbm.at[pl.ds(w * SEG_PER_W, SEG_PER_W), :])


def _tc_body(ids_ref, x_ref, out_ref, acc_ref):
    i = pl.program_id(0)

    @pl.when(i == 0)
    def _():
        acc_ref[...] = jnp.zeros_like(acc_ref)

    ids2 = ids_ref[0]
    xb16 = x_ref[...].astype(jnp.bfloat16)
    first = ids2[0, 0]
    last = ids2[0, RB - 1]
    s0a = pl.multiple_of(jnp.minimum((first // 8) * 8, S - W), 8)
    span_ok = (last - first) <= (W - 8)

    @pl.when(span_ok)
    def _():
        iota_w = lax.broadcasted_iota(jnp.int32, (W, RB), 0) + s0a
        ohw = (iota_w == jnp.broadcast_to(ids2, (W, RB))).astype(jnp.bfloat16)
        pw = lax.dot_general(
            ohw, xb16, (((1,), (0,)), ((), ())),
            preferred_element_type=jnp.float32)
        acc_ref[pl.ds(s0a, W), :] += pw

    @pl.when(jnp.logical_not(span_ok))
    def _():
        iota = lax.broadcasted_iota(jnp.int32, (S, RB), 0)
        oh = (iota == jnp.broadcast_to(ids2, (S, RB))).astype(jnp.bfloat16)
        acc_ref[...] += lax.dot_general(
            oh, xb16, (((1,), (0,)), ((), ())),
            preferred_element_type=jnp.float32)

    @pl.when(i == NB - 1)
    def _():
        out_ref[...] = acc_ref[...]


def _combine_body(a_ref, b_ref, cnt_ref, out_ref):
    out_ref[...] = (a_ref[...] + b_ref[...]) / jnp.maximum(cnt_ref[...], 1.0)


def kernel(x, segment_ids):
    ids32 = segment_ids.astype(jnp.int32)
    offs = jnp.searchsorted(
        ids32, jnp.arange(S + 1, dtype=jnp.int32), side="left",
        method="compare_all").astype(jnp.int32)
    counts = (offs[1:] - offs[:-1]).astype(jnp.float32).reshape(S, 1)
    offs_p = jnp.pad(offs, (0, OFFS_PAD - (S + 1)))

    mesh = plsc.VectorSubcoreMesh(core_axis_name="c", subcore_axis_name="s")
    cp = pltpu.CompilerParams()
    if "needs_layout_passes" in pltpu.CompilerParams.__dataclass_fields__:
        cp = dataclasses.replace(cp, needs_layout_passes=False)
    sc_fn = pl.kernel(
        _sc_body,
        out_type=jax.ShapeDtypeStruct((S, D), jnp.float32),
        mesh=mesh,
        compiler_params=cp,
        scratch_types=[
            pltpu.VMEM((CH, D), jnp.float32),
            pltpu.VMEM((CH, D), jnp.float32),
            pltpu.VMEM((OFFS_PAD,), jnp.int32),
            pltpu.VMEM((SEG_PER_W, D), jnp.float32),
            pltpu.SemaphoreType.DMA,
            pltpu.SemaphoreType.DMA,
        ],
    )
    sums_sc = sc_fn(x, offs_p)

    ids3 = ids32[:RT].reshape(NB, 1, RB)
    sums_tc = pl.pallas_call(
        _tc_body,
        grid=(NB,),
        in_specs=[
            pl.BlockSpec((1, 1, RB), lambda i: (i, 0, 0)),
            pl.BlockSpec((RB, D), lambda i: (i, 0)),
        ],
        out_specs=pl.BlockSpec((S, D), lambda i: (0, 0)),
        out_shape=jax.ShapeDtypeStruct((S, D), jnp.float32),
        scratch_shapes=[pltpu.VMEM((S, D), jnp.float32)],
    )(ids3, x)

    return pl.pallas_call(
        _combine_body,
        out_shape=jax.ShapeDtypeStruct((S, D), jnp.float32),
    )(sums_tc, sums_sc, counts)


# speedup vs baseline: 1.0299x; 1.0299x over previous
import dataclasses

import jax
import jax.numpy as jnp
from jax import lax
from jax.experimental import pallas as pl
from jax.experimental.pallas import tpu as pltpu
from jax.experimental.pallas import tpu_sc as plsc

N_ROWS = 100000
D = 512
S = 256
NC = 2
NS = 16
NW = NC * NS
LANES = 16

RT = 53000
CH = 96
FH = 2
DHF = D // FH
FVEC = DHF // LANES
SEG_PER_W = S // NW
OFFS_PAD = 272

RB = 1000
NB = RT // RB
W = 32


def _sc_body(x_hbm, offs_hbm, out_hbm, xb0, xb1, offs_v, acc_v, sem0, sem1):
    c = lax.axis_index("c")
    s = lax.axis_index("s")
    w = s * NC + c

    zero = jnp.zeros((LANES,), jnp.float32)
    lane = lax.iota(jnp.int32, 16)

    pltpu.sync_copy(offs_hbm, offs_v)

    w0 = offs_v[pl.ds(w * SEG_PER_W, LANES)]
    w1 = offs_v[pl.ds(w * SEG_PER_W + 1, LANES)]

    def extract(vec, t):
        return jnp.maximum(jnp.sum(jnp.where(lane == t, vec, 0)), RT)

    seg_lo = [extract(w0, t) for t in range(SEG_PER_W)]
    seg_hi = [extract(w1, t) for t in range(SEG_PER_W)]
    row_start = seg_lo[0]
    row_end = seg_hi[SEG_PER_W - 1]

    @pl.loop(0, SEG_PER_W)
    def _(i):
        @pl.loop(0, FH * FVEC)
        def _(f):
            acc_v[i, pl.ds(f * LANES, LANES)] = zero

    a8 = (row_start // 8) * 8
    nk = (row_end - a8 + CH - 1) // CH

    def chunk_base(k):
        return jnp.minimum(a8 + k * CH, N_ROWS - CH)

    def start(k, xb, sem):
        pltpu.async_copy(x_hbm.at[pl.ds(chunk_base(k), CH), :], xb, sem)

    def wait(xb, sem):
        pltpu.make_async_copy(x_hbm.at[pl.ds(0, CH), :], xb, sem).wait()

    def process(k, xb):
        base = chunk_base(k)
        win_lo = jnp.maximum(row_start, a8 + k * CH)
        win_hi = jnp.minimum(row_end, a8 + (k + 1) * CH)
        for t in range(SEG_PER_W):
            a = jnp.maximum(seg_lo[t], win_lo) - base
            b = jnp.minimum(seg_hi[t], win_hi) - base
            for h in range(FH):
                def row_body(r, regs, h=h):
                    return tuple(
                        regs[f] + xb[r, pl.ds(h * DHF + f * LANES, LANES)]
                        for f in range(FVEC))

                regs0 = tuple(zero for _ in range(FVEC))
                regs = lax.fori_loop(a, b, row_body, regs0)
                for f in range(FVEC):
                    plsc.addupdate(
                        acc_v.at[t, pl.ds(h * DHF + f * LANES, LANES)],
                        regs[f])

    @pl.when(nk > 0)
    def _():
        start(0, xb0, sem0)

    def pair_body(m, carry):
        k0 = 2 * m

        @pl.when(k0 + 1 < nk)
        def _():
            start(k0 + 1, xb1, sem1)

        wait(xb0, sem0)
        process(k0, xb0)

        @pl.when(k0 + 2 < nk)
        def _():
            start(k0 + 2, xb0, sem0)

        @pl.when(k0 + 1 < nk)
        def _():
            wait(xb1, sem1)
            process(k0 + 1, xb1)

        return carry

    lax.fori_loop(0, (nk + 1) // 2, pair_body, 0)

    pltpu.sync_copy(acc_v, out_hbm.at[pl.ds(w * SEG_PER_W, SEG_PER_W), :])


def _tc_body(ids_ref, x_ref, out_ref, acc_ref):
    i = pl.program_id(0)

    @pl.when(i == 0)
    def _():
        acc_ref[...] = jnp.zeros_like(acc_ref)

    ids2 = ids_ref[0]
    xb16 = x_ref[...].astype(jnp.bfloat16)
    first = ids2[0, 0]
    last = ids2[0, RB - 1]
    s0a = pl.multiple_of(jnp.minimum((first // 8) * 8, S - W), 8)
    span_ok = (last - first) <= (W - 8)

    @pl.when(span_ok)
    def _():
        iota_w = lax.broadcasted_iota(jnp.int32, (W, RB), 0) + s0a
        ohw = (iota_w == jnp.broadcast_to(ids2, (W, RB))).astype(jnp.bfloat16)
        pw = lax.dot_general(
            ohw, xb16, (((1,), (0,)), ((), ())),
            preferred_element_type=jnp.float32)
        acc_ref[pl.ds(s0a, W), :] += pw

    @pl.when(jnp.logical_not(span_ok))
    def _():
        iota = lax.broadcasted_iota(jnp.int32, (S, RB), 0)
        oh = (iota == jnp.broadcast_to(ids2, (S, RB))).astype(jnp.bfloat16)
        acc_ref[...] += lax.dot_general(
            oh, xb16, (((1,), (0,)), ((), ())),
            preferred_element_type=jnp.float32)

    @pl.when(i == NB - 1)
    def _():
        out_ref[...] = acc_ref[...]


def _combine_body(a_ref, b_ref, cnt_ref, out_ref):
    out_ref[...] = (a_ref[...] + b_ref[...]) / jnp.maximum(cnt_ref[...], 1.0)


def kernel(x, segment_ids):
    ids32 = segment_ids.astype(jnp.int32)
    offs = jnp.searchsorted(
        ids32, jnp.arange(S + 1, dtype=jnp.int32), side="left",
        method="compare_all").astype(jnp.int32)
    counts = (offs[1:] - offs[:-1]).astype(jnp.float32).reshape(S, 1)
    offs_p = jnp.pad(offs, (0, OFFS_PAD - (S + 1)))

    mesh = plsc.VectorSubcoreMesh(core_axis_name="c", subcore_axis_name="s")
    cp = pltpu.CompilerParams()
    if "needs_layout_passes" in pltpu.CompilerParams.__dataclass_fields__:
        cp = dataclasses.replace(cp, needs_layout_passes=False)
    sc_fn = pl.kernel(
        _sc_body,
        out_type=jax.ShapeDtypeStruct((S, D), jnp.float32),
        mesh=mesh,
        compiler_params=cp,
        scratch_types=[
            pltpu.VMEM((CH, D), jnp.float32),
            pltpu.VMEM((CH, D), jnp.float32),
            pltpu.VMEM((OFFS_PAD,), jnp.int32),
            pltpu.VMEM((SEG_PER_W, D), jnp.float32),
            pltpu.SemaphoreType.DMA,
            pltpu.SemaphoreType.DMA,
        ],
    )
    sums_sc = sc_fn(x, offs_p)

    ids3 = ids32[:RT].reshape(NB, 1, RB)
    sums_tc = pl.pallas_call(
        _tc_body,
        grid=(NB,),
        in_specs=[
            pl.BlockSpec((1, 1, RB), lambda i: (i, 0, 0)),
            pl.BlockSpec((RB, D), lambda i: (i, 0)),
        ],
        out_specs=pl.BlockSpec((S, D), lambda i: (0, 0)),
        out_shape=jax.ShapeDtypeStruct((S, D), jnp.float32),
        scratch_shapes=[pltpu.VMEM((S, D), jnp.float32)],
    )(ids3, x)

    return pl.pallas_call(
        _combine_body,
        out_shape=jax.ShapeDtypeStruct((S, D), jnp.float32),
    )(sums_tc, sums_sc, counts)
